# Initial kernel scaffold; baseline (speedup 1.0000x reference)
import jax
import jax.numpy as jnp
from jax.experimental import pallas as pl


def kernel(state, action, x, edge_index, W_g1, b_g1, W_g2, b_g2, Wa0, ba0, Wa1, ba1, Wa2, ba2, Wc0, bc0, Wc1, bc1, Wc2, bc2):
    raise NotImplementedError("write your pallas kernel here")


# speedup vs baseline: 45.7107x; 45.7107x over previous
import functools

import jax
import jax.numpy as jnp
from jax import lax
from jax.experimental import pallas as pl
from jax.experimental.pallas import tpu as pltpu
from jax.experimental.pallas import tpu_sc as plsc

_N = 10000
_E = 320000
_NT = 16
_EPW = _E // _NT
_SLICE = 640
_NPAD = _NT * _SLICE
_L = 16

_HI = jax.lax.Precision.HIGHEST


def _rsqrt16(x):
    i = plsc.bitcast(x, jnp.int32)
    i = jnp.int32(0x5F3759DF) - lax.shift_right_logical(i, 1)
    y = plsc.bitcast(i, jnp.float32)
    for _ in range(4):
        y = y * (1.5 - 0.5 * x * y * y)
    return y


def _sc_body(src_hbm, dst_hbm, w_hbm, s1_hbm,
             src_v, dst_v, acc_v, dinv_v, u_v, comb_v, stage_v, s1_v,
             shacc, shtbl):
    wid = lax.axis_index("s")
    ebase = wid * _EPW
    nbase = wid * _SLICE

    pltpu.sync_copy(src_hbm.at[pl.ds(ebase, _EPW)], src_v)
    pltpu.sync_copy(dst_hbm.at[pl.ds(ebase, _EPW)], dst_v)

    zeros16 = jnp.zeros((_L,), jnp.float32)
    ones16 = jnp.ones((_L,), jnp.float32)
    lane = lax.broadcasted_iota(jnp.int32, (_L,), 0)

    def zero_acc():
        def zbody(i, _):
            acc_v[pl.ds(i * _L, _L)] = zeros16
            return ()
        lax.fori_loop(0, _NPAD // _L, zbody, ())

    def edge_pass(gather_tbl):
        def ebody(i, _):
            off = i * _L
            d = dst_v[pl.ds(off, _L)]
            if gather_tbl is None:
                vals = ones16
            else:
                vals = plsc.load_gather(gather_tbl, [d])
            s = src_v[pl.ds(off, _L)]
            plsc.addupdate_scatter(acc_v, [s], vals)
            return ()
        lax.fori_loop(0, _EPW // _L, ebody, ())

    def combine_rows():
        for k in range(_NT):
            pltpu.sync_copy(shacc.at[k, pl.ds(nbase, _SLICE)], comb_v.at[k])

    def comb_chunk(j):
        t = comb_v[0, pl.ds(j * _L, _L)]
        for k in range(1, _NT):
            t = t + comb_v[k, pl.ds(j * _L, _L)]
        return t

    zero_acc()

    def deg_body(i, _):
        d = dst_v[pl.ds(i * _L, _L)]
        plsc.addupdate_scatter(acc_v, [d], ones16)
        return ()
    lax.fori_loop(0, _EPW // _L, deg_body, ())

    pltpu.sync_copy(acc_v, shacc.at[wid])
    plsc.subcore_barrier()

    combine_rows()

    def dinv_body(j, _):
        deg = comb_chunk(j) + 1.0
        deg = jnp.maximum(deg, 1.0)
        stage_v[pl.ds(j * _L, _L)] = _rsqrt16(deg)
        return ()
    lax.fori_loop(0, _SLICE // _L, dinv_body, ())

    pltpu.sync_copy(stage_v, shtbl.at[pl.ds(nbase, _SLICE)])
    plsc.subcore_barrier()
    pltpu.sync_copy(shtbl, dinv_v)

    zero_acc()
    edge_pass(dinv_v)
    pltpu.sync_copy(acc_v, shacc.at[wid])
    plsc.subcore_barrier()

    combine_rows()
    s1_v[...] = zeros16

    def v_body(j, _):
        a = comb_chunk(j)
        dv = dinv_v[pl.ds(nbase + j * _L, _L)]
        v = dv * (dv + a)
        gidx = nbase + j * _L + lane
        v = jnp.where(gidx < _N, v, 0.0)
        s1_v[...] = s1_v[...] + v
        stage_v[pl.ds(j * _L, _L)] = dv * v
        return ()
    lax.fori_loop(0, _SLICE // _L, v_body, ())

    pltpu.sync_copy(s1_v, s1_hbm.at[wid])
    pltpu.sync_copy(stage_v, shtbl.at[pl.ds(nbase, _SLICE)])
    plsc.subcore_barrier()
    pltpu.sync_copy(shtbl, u_v)

    zero_acc()
    edge_pass(u_v)
    pltpu.sync_copy(acc_v, shacc.at[wid])
    plsc.subcore_barrier()

    combine_rows()

    def w_body(j, _):
        a = comb_chunk(j)
        dv = dinv_v[pl.ds(nbase + j * _L, _L)]
        u = u_v[pl.ds(nbase + j * _L, _L)]
        w = dv * (u + a)
        gidx = nbase + j * _L + lane
        w = jnp.where(gidx < _N, w, 0.0)
        stage_v[pl.ds(j * _L, _L)] = w
        return ()
    lax.fori_loop(0, _SLICE // _L, w_body, ())

    pltpu.sync_copy(stage_v, w_hbm.at[pl.ds(nbase, _SLICE)])


def _sc_edge_kernel(src, dst):
    return pl.kernel(
        _sc_body,
        out_type=(
            jax.ShapeDtypeStruct((_NPAD,), jnp.float32),
            jax.ShapeDtypeStruct((_NT, _L), jnp.float32),
        ),
        mesh=plsc.VectorSubcoreMesh(
            core_axis_name="c", subcore_axis_name="s",
            num_cores=1, num_subcores=_NT,
        ),
        scratch_types=[
            pltpu.VMEM((_EPW,), jnp.int32),
            pltpu.VMEM((_EPW,), jnp.int32),
            pltpu.VMEM((_NPAD,), jnp.float32),
            pltpu.VMEM((_NPAD,), jnp.float32),
            pltpu.VMEM((_NPAD,), jnp.float32),
            pltpu.VMEM((_NT, _SLICE), jnp.float32),
            pltpu.VMEM((_SLICE,), jnp.float32),
            pltpu.VMEM((_L,), jnp.float32),
            pltpu.VMEM_SHARED((_NT, _NPAD), jnp.float32),
            pltpu.VMEM_SHARED((_NPAD,), jnp.float32),
        ],
        compiler_params=pltpu.CompilerParams(needs_layout_passes=False),
    )(src, dst)


def _tc_body(wv_ref, x_ref, s1_ref, state_ref, act_ref,
             wg1_ref, bg1_ref, wg2_ref, bg2_ref,
             wa0s_ref, wa0g_ref, ba0_ref, wa1_ref, ba1_ref, wa2_ref, ba2_ref,
             wc0s_ref, wc0g_ref, bc0_ref, wc1_ref, bc1_ref, wc2_ref, bc2_ref,
             alp_ref, sv_ref, ent_ref):
    r = jnp.dot(wv_ref[...], x_ref[...], precision=_HI)
    s1 = jnp.sum(s1_ref[...])
    t = jnp.dot(r, wg1_ref[...], precision=_HI) + s1 * bg1_ref[...]
    g = jnp.dot(t, wg2_ref[...], precision=_HI) * (1.0 / _N) + bg2_ref[...]

    state = state_ref[...]

    def head(ws_ref, wg_ref, b0_ref, w1_ref, b1_ref, w2_ref, b2_ref):
        pre = (jnp.dot(state, ws_ref[...], precision=_HI)
               + jnp.dot(g, wg_ref[...], precision=_HI) + b0_ref[...])
        y = jnp.tanh(pre)
        y = jnp.tanh(jnp.dot(y, w1_ref[...], precision=_HI) + b1_ref[...])
        return jnp.dot(y, w2_ref[...], precision=_HI) + b2_ref[...]

    logits = head(wa0s_ref, wa0g_ref, ba0_ref, wa1_ref, ba1_ref,
                  wa2_ref, ba2_ref)
    m = jnp.max(logits, axis=1, keepdims=True)
    ex = jnp.exp(logits - m)
    se = jnp.sum(ex, axis=1, keepdims=True)
    logp = logits - m - jnp.log(se)
    onehot = lax.broadcasted_iota(jnp.int32, logits.shape, 1) == act_ref[...]
    alp_ref[...] = jnp.sum(jnp.where(onehot, logp, 0.0), axis=1, keepdims=True)
    probs = ex / se
    ent_ref[...] = -jnp.sum(probs * logp, axis=1, keepdims=True)

    sv_ref[...] = head(wc0s_ref, wc0g_ref, bc0_ref, wc1_ref, bc1_ref,
                       wc2_ref, bc2_ref)


def kernel(state, action, x, edge_index, W_g1, b_g1, W_g2, b_g2,
           Wa0, ba0, Wa1, ba1, Wa2, ba2, Wc0, bc0, Wc1, bc1, Wc2, bc2):
    src = edge_index[0]
    dst = edge_index[1]

    w_pad, s1p = _sc_edge_kernel(src, dst)
    wv = w_pad[:_N].reshape(1, _N)

    B = state.shape[0]
    ACT = Wa2.shape[1]
    STATE = state.shape[1]

    out_shapes = (
        jax.ShapeDtypeStruct((B, 1), jnp.float32),
        jax.ShapeDtypeStruct((B, 1), jnp.float32),
        jax.ShapeDtypeStruct((B, 1), jnp.float32),
    )
    alp, sv, ent = pl.pallas_call(_tc_body, out_shape=out_shapes)(
        wv, x, s1p, state, action.reshape(B, 1),
        W_g1, b_g1.reshape(1, -1), W_g2, b_g2.reshape(1, -1),
        Wa0[:STATE], Wa0[STATE:], ba0.reshape(1, -1), Wa1, ba1.reshape(1, -1),
        Wa2, ba2.reshape(1, -1),
        Wc0[:STATE], Wc0[STATE:], bc0.reshape(1, -1), Wc1, bc1.reshape(1, -1),
        Wc2, bc2.reshape(1, -1),
    )
    return alp[:, 0], sv, ent[:, 0]


# speedup vs baseline: 57.0249x; 1.2475x over previous
import functools

import jax
import jax.numpy as jnp
from jax import lax
from jax.experimental import pallas as pl
from jax.experimental.pallas import tpu as pltpu
from jax.experimental.pallas import tpu_sc as plsc

_N = 10000
_E = 320000
_NT = 16
_EPW = _E // _NT
_SLICE = 640
_NPAD = _NT * _SLICE
_L = 16

_HI = jax.lax.Precision.HIGHEST


def _rsqrt16(x):
    i = plsc.bitcast(x, jnp.int32)
    i = jnp.int32(0x5F3759DF) - lax.shift_right_logical(i, 1)
    y = plsc.bitcast(i, jnp.float32)
    for _ in range(4):
        y = y * (1.5 - 0.5 * x * y * y)
    return y


def _sc_body(edge_hbm, w_hbm, s1_hbm,
             src_v, dst_v, acc_v, dinv_v, u_v, comb_v, stage_v, s1_v,
             shacc, shtbl):
    wid = lax.axis_index("s")
    ebase = wid * _EPW
    nbase = wid * _SLICE

    pltpu.sync_copy(edge_hbm.at[pl.ds(ebase, _EPW)], src_v)
    pltpu.sync_copy(edge_hbm.at[pl.ds(_E + ebase, _EPW)], dst_v)

    zeros16 = jnp.zeros((_L,), jnp.float32)
    ones16 = jnp.ones((_L,), jnp.float32)
    lane = lax.broadcasted_iota(jnp.int32, (_L,), 0)

    def zero_acc():
        def zbody(i, _):
            for u in range(8):
                acc_v[pl.ds(i * (8 * _L) + u * _L, _L)] = zeros16
            return ()
        lax.fori_loop(0, _NPAD // (8 * _L), zbody, ())

    def edge_pass(gather_tbl):
        def ebody(i, _):
            base = i * (5 * _L)
            for u in range(5):
                off = base + u * _L
                d = dst_v[pl.ds(off, _L)]
                vals = plsc.load_gather(gather_tbl, [d])
                s = src_v[pl.ds(off, _L)]
                plsc.addupdate_scatter(acc_v, [s], vals)
            return ()
        lax.fori_loop(0, _EPW // (5 * _L), ebody, ())

    def combine_rows():
        pltpu.sync_copy(shacc.at[:, pl.ds(nbase, _SLICE)], comb_v)

    def comb_chunk(j):
        t = comb_v[0, pl.ds(j * _L, _L)]
        for k in range(1, _NT):
            t = t + comb_v[k, pl.ds(j * _L, _L)]
        return t

    zero_acc()

    def deg_body(i, _):
        base = i * (5 * _L)
        for u in range(5):
            d = dst_v[pl.ds(base + u * _L, _L)]
            plsc.addupdate_scatter(acc_v, [d], ones16)
        return ()
    lax.fori_loop(0, _EPW // (5 * _L), deg_body, ())

    pltpu.sync_copy(acc_v, shacc.at[wid])
    plsc.subcore_barrier()

    combine_rows()

    def dinv_body(j, _):
        deg = comb_chunk(j) + 1.0
        deg = jnp.maximum(deg, 1.0)
        stage_v[pl.ds(j * _L, _L)] = _rsqrt16(deg)
        return ()
    lax.fori_loop(0, _SLICE // _L, dinv_body, ())

    pltpu.sync_copy(stage_v, shtbl.at[pl.ds(nbase, _SLICE)])
    plsc.subcore_barrier()
    pltpu.sync_copy(shtbl, dinv_v)

    zero_acc()
    edge_pass(dinv_v)
    pltpu.sync_copy(acc_v, shacc.at[wid])
    plsc.subcore_barrier()

    combine_rows()
    s1_v[...] = zeros16

    def v_body(j, _):
        a = comb_chunk(j)
        dv = dinv_v[pl.ds(nbase + j * _L, _L)]
        v = dv * (dv + a)
        gidx = nbase + j * _L + lane
        v = jnp.where(gidx < _N, v, 0.0)
        s1_v[...] = s1_v[...] + v
        stage_v[pl.ds(j * _L, _L)] = dv * v
        return ()
    lax.fori_loop(0, _SLICE // _L, v_body, ())

    pltpu.sync_copy(s1_v, s1_hbm.at[wid])
    pltpu.sync_copy(stage_v, shtbl.at[pl.ds(nbase, _SLICE)])
    plsc.subcore_barrier()
    pltpu.sync_copy(shtbl, u_v)

    zero_acc()
    edge_pass(u_v)
    pltpu.sync_copy(acc_v, shacc.at[wid])
    plsc.subcore_barrier()

    combine_rows()

    def w_body(j, _):
        a = comb_chunk(j)
        dv = dinv_v[pl.ds(nbase + j * _L, _L)]
        u = u_v[pl.ds(nbase + j * _L, _L)]
        w = dv * (u + a)
        gidx = nbase + j * _L + lane
        w = jnp.where(gidx < _N, w, 0.0)
        stage_v[pl.ds(j * _L, _L)] = w
        return ()
    lax.fori_loop(0, _SLICE // _L, w_body, ())

    pltpu.sync_copy(stage_v, w_hbm.at[pl.ds(nbase, _SLICE)])


def _sc_edge_kernel(edge_index):
    return pl.kernel(
        _sc_body,
        out_type=(
            jax.ShapeDtypeStruct((_NPAD,), jnp.float32),
            jax.ShapeDtypeStruct((_NT, _L), jnp.float32),
        ),
        mesh=plsc.VectorSubcoreMesh(
            core_axis_name="c", subcore_axis_name="s",
            num_cores=1, num_subcores=_NT,
        ),
        scratch_types=[
            pltpu.VMEM((_EPW,), jnp.int32),
            pltpu.VMEM((_EPW,), jnp.int32),
            pltpu.VMEM((_NPAD,), jnp.float32),
            pltpu.VMEM((_NPAD,), jnp.float32),
            pltpu.VMEM((_NPAD,), jnp.float32),
            pltpu.VMEM((_NT, _SLICE), jnp.float32),
            pltpu.VMEM((_SLICE,), jnp.float32),
            pltpu.VMEM((_L,), jnp.float32),
            pltpu.VMEM_SHARED((_NT, _NPAD), jnp.float32),
            pltpu.VMEM_SHARED((_NPAD,), jnp.float32),
        ],
        compiler_params=pltpu.CompilerParams(needs_layout_passes=False),
    )(edge_index)


def _tc_body(wv_ref, x_ref, s1_ref, state_ref, act_ref,
             wg1_ref, bg1_ref, wg2_ref, bg2_ref,
             wa0s_ref, wa0g_ref, ba0_ref, wa1_ref, ba1_ref, wa2_ref, ba2_ref,
             wc0s_ref, wc0g_ref, bc0_ref, wc1_ref, bc1_ref, wc2_ref, bc2_ref,
             alp_ref, sv_ref, ent_ref):
    r = jnp.dot(wv_ref[...], x_ref[...], precision=_HI)
    s1 = jnp.sum(s1_ref[...])
    t = jnp.dot(r, wg1_ref[...], precision=_HI) + s1 * bg1_ref[...]
    g = jnp.dot(t, wg2_ref[...], precision=_HI) * (1.0 / _N) + bg2_ref[...]

    state = state_ref[...]

    def head(ws_ref, wg_ref, b0_ref, w1_ref, b1_ref, w2_ref, b2_ref):
        pre = (jnp.dot(state, ws_ref[...], precision=_HI)
               + jnp.dot(g, wg_ref[...], precision=_HI) + b0_ref[...])
        y = jnp.tanh(pre)
        y = jnp.tanh(jnp.dot(y, w1_ref[...], precision=_HI) + b1_ref[...])
        return jnp.dot(y, w2_ref[...], precision=_HI) + b2_ref[...]

    logits = head(wa0s_ref, wa0g_ref, ba0_ref, wa1_ref, ba1_ref,
                  wa2_ref, ba2_ref)
    m = jnp.max(logits, axis=1, keepdims=True)
    ex = jnp.exp(logits - m)
    se = jnp.sum(ex, axis=1, keepdims=True)
    logp = logits - m - jnp.log(se)
    onehot = lax.broadcasted_iota(jnp.int32, logits.shape, 1) == act_ref[...]
    alp_ref[...] = jnp.sum(jnp.where(onehot, logp, 0.0), axis=1, keepdims=True)
    probs = ex / se
    ent_ref[...] = -jnp.sum(probs * logp, axis=1, keepdims=True)

    sv_ref[...] = head(wc0s_ref, wc0g_ref, bc0_ref, wc1_ref, bc1_ref,
                       wc2_ref, bc2_ref)


def kernel(state, action, x, edge_index, W_g1, b_g1, W_g2, b_g2,
           Wa0, ba0, Wa1, ba1, Wa2, ba2, Wc0, bc0, Wc1, bc1, Wc2, bc2):
    w_pad, s1p = _sc_edge_kernel(edge_index.reshape(-1))
    wv = w_pad[:_N].reshape(1, _N)

    B = state.shape[0]
    ACT = Wa2.shape[1]
    STATE = state.shape[1]

    out_shapes = (
        jax.ShapeDtypeStruct((B, 1), jnp.float32),
        jax.ShapeDtypeStruct((B, 1), jnp.float32),
        jax.ShapeDtypeStruct((B, 1), jnp.float32),
    )
    alp, sv, ent = pl.pallas_call(_tc_body, out_shape=out_shapes)(
        wv, x, s1p, state, action.reshape(B, 1),
        W_g1, b_g1.reshape(1, -1), W_g2, b_g2.reshape(1, -1),
        Wa0[:STATE], Wa0[STATE:], ba0.reshape(1, -1), Wa1, ba1.reshape(1, -1),
        Wa2, ba2.reshape(1, -1),
        Wc0[:STATE], Wc0[STATE:], bc0.reshape(1, -1), Wc1, bc1.reshape(1, -1),
        Wc2, bc2.reshape(1, -1),
    )
    return alp[:, 0], sv, ent[:, 0]
